# baseline (speedup 1.0000x reference)
import jax
import jax.numpy as jnp
from jax import lax
from jax.experimental import pallas as pl
from jax.experimental.pallas import tpu as pltpu
from jax.experimental.pallas import tpu_sc as plsc

N_NODES = 10000
N_EDGES = 320000
D = 128
N_TILES = 32
NPT = 320
N_PAD = N_TILES * NPT
CHUNK = 2560
N_CHUNKS = N_EDGES // CHUNK
GB = 128
NW = 3
FBLK = 32


def _mm_body(d_ref, w_ref, b_ref, o_ref):
    o_ref[...] = jnp.maximum(
        jnp.dot(d_ref[...], w_ref[...], preferred_element_type=jnp.float32)
        + b_ref[...],
        0.0,
    )


def _encode(data_pad, W, b2):
    blk = 2048
    return pl.pallas_call(
        _mm_body,
        grid=(N_PAD // blk,),
        in_specs=[
            pl.BlockSpec((blk, D), lambda i: (i, 0)),
            pl.BlockSpec((D, D), lambda i: (0, 0)),
            pl.BlockSpec((1, D), lambda i: (0, 0)),
        ],
        out_specs=pl.BlockSpec((blk, D), lambda i: (i, 0)),
        out_shape=jax.ShapeDtypeStruct((N_PAD, D), jnp.float32),
    )(data_pad, W, b2)


def _edge_body(x_hbm, src_hbm, dst_hbm, out1_hbm, out2_hbm,
               acc, dstc, srcc, kdst, ksrc, rows, dring, sring,
               xblk, o1blk, o2blk, sem_d, sem_s, sem_g):
    c = lax.axis_index("c")
    s = lax.axis_index("s")
    t = s * 2 + c
    lo = t * NPT

    iota = lax.iota(jnp.int32, 16)
    neg = jnp.full((16,), -1.0, jnp.float32)

    def init_row(r, _):
        for v in range(8):
            acc[r, pl.ds(v * 16, 16)] = neg
        return 0

    lax.fori_loop(0, NPT + 1, init_row, 0)

    def fire_edges(ci):
        base = ci * CHUNK
        buf = lax.rem(ci, 2)
        pltpu.async_copy(dst_hbm.at[pl.ds(base, CHUNK)], dstc.at[buf], sem_d)
        pltpu.async_copy(src_hbm.at[pl.ds(base, CHUNK)], srcc.at[buf], sem_s)

    fire_edges(0)

    def drain(G):
        slot = lax.rem(G, NW)
        pltpu.make_async_copy(
            x_hbm.at[sring.at[slot]], rows.at[slot], sem_g.at[slot]).wait()
        slot_v = jnp.broadcast_to(slot, (16,))

        def grp(g, _):
            e0 = g * 16
            for j in range(16):
                djv = plsc.load_gather(
                    dring, [slot_v, jnp.broadcast_to(e0 + j, (16,))])
                for v in range(8):
                    col = iota + v * 16
                    old = plsc.load_gather(acc, [djv, col])
                    new = jnp.maximum(old, rows[slot, e0 + j,
                                                pl.ds(v * 16, 16)])
                    plsc.store_scatter(acc, [djv, col], new)
            return 0

        lax.fori_loop(0, GB // 16, grp, 0)

    def chunk_body(ci, F):
        buf = lax.rem(ci, 2)
        base = ci * CHUNK
        pltpu.make_async_copy(
            dst_hbm.at[pl.ds(base, CHUNK)], dstc.at[buf], sem_d).wait()
        pltpu.make_async_copy(
            src_hbm.at[pl.ds(base, CHUNK)], srcc.at[buf], sem_s).wait()

        @pl.when(ci + 1 < N_CHUNKS)
        def _():
            fire_edges(ci + 1)

        def filt(i, off):
            dvec = dstc[buf, pl.ds(i * 16, 16)]
            svec = srcc[buf, pl.ds(i * 16, 16)]
            dloc = dvec - lo
            m = (dloc >= 0) & (dloc < NPT)
            mi = m.astype(jnp.int32)
            pos = off + plsc.cumsum(mi) - 1
            plsc.store_scatter(kdst, [pos], dloc, mask=m)
            plsc.store_scatter(ksrc, [pos], svec, mask=m)
            return off + plsc.all_reduce_population_count(m)

        off = lax.fori_loop(0, CHUNK // 16, filt, jnp.zeros((16,), jnp.int32))
        k = off[0]

        for j in range(GB // 16):
            kdst[pl.ds(k + j * 16, 16)] = jnp.full((16,), NPT, jnp.int32)
            ksrc[pl.ds(k + j * 16, 16)] = jnp.zeros((16,), jnp.int32)
        nb = (k + GB - 1) // GB

        def batch(b, F):
            B = F + b
            slot = lax.rem(B, NW)

            @pl.when(B >= NW)
            def _():
                drain(B - NW)

            def stage(i, _):
                dring[slot, pl.ds(i * 16, 16)] = kdst[pl.ds(b * GB + i * 16,
                                                            16)]
                sring[slot, pl.ds(i * 16, 16)] = ksrc[pl.ds(b * GB + i * 16,
                                                            16)]
                return 0

            lax.fori_loop(0, GB // 16, stage, 0)
            pltpu.async_copy(
                x_hbm.at[sring.at[slot]], rows.at[slot], sem_g.at[slot])
            return F

        lax.fori_loop(0, nb, batch, F)
        return F + nb

    F = lax.fori_loop(0, N_CHUNKS, chunk_body, jnp.int32(0))

    for w in range(NW):
        G = F - NW + w

        @pl.when(G >= 0)
        def _():
            drain(G)

    def fin(bi, _):
        r0 = lo + bi * FBLK
        pltpu.sync_copy(x_hbm.at[pl.ds(r0, FBLK)], xblk)

        def frow(r, _):
            ar = bi * FBLK + r
            for v in range(8):
                sl = pl.ds(v * 16, 16)
                a = acc[ar, sl]
                xv = xblk[r, sl]
                valid = a >= 0.0
                o1blk[r, sl] = jnp.where(valid, xv, 0.0)
                o2blk[r, sl] = jnp.where(valid, a - xv, 0.0)
            return 0

        lax.fori_loop(0, FBLK, frow, 0)
        pltpu.sync_copy(o1blk, out1_hbm.at[pl.ds(r0, FBLK)])
        pltpu.sync_copy(o2blk, out2_hbm.at[pl.ds(r0, FBLK)])
        return 0

    lax.fori_loop(0, NPT // FBLK, fin, 0)


_edge_call = pl.kernel(
    _edge_body,
    out_type=[
        jax.ShapeDtypeStruct((N_PAD, D), jnp.float32),
        jax.ShapeDtypeStruct((N_PAD, D), jnp.float32),
    ],
    mesh=plsc.VectorSubcoreMesh(core_axis_name="c", subcore_axis_name="s"),
    compiler_params=pltpu.CompilerParams(needs_layout_passes=False),
    scratch_types=[
        pltpu.VMEM((NPT + 1, D), jnp.float32),
        pltpu.VMEM((2, CHUNK), jnp.int32),
        pltpu.VMEM((2, CHUNK), jnp.int32),
        pltpu.VMEM((CHUNK + GB,), jnp.int32),
        pltpu.VMEM((CHUNK + GB,), jnp.int32),
        pltpu.VMEM((NW, GB, D), jnp.float32),
        pltpu.VMEM((NW, GB), jnp.int32),
        pltpu.VMEM((NW, GB), jnp.int32),
        pltpu.VMEM((FBLK, D), jnp.float32),
        pltpu.VMEM((FBLK, D), jnp.float32),
        pltpu.VMEM((FBLK, D), jnp.float32),
        pltpu.SemaphoreType.DMA,
        pltpu.SemaphoreType.DMA,
        pltpu.SemaphoreType.DMA((NW,)),
    ],
)


def kernel(data, edge_index, W, b):
    data_pad = jnp.pad(data, ((0, N_PAD - N_NODES), (0, 0)))
    x = _encode(data_pad, W, b.reshape(1, D))
    src = edge_index[0]
    dst = edge_index[1]
    out1, out2 = _edge_call(x, src, dst)
    return jnp.concatenate([out1[:N_NODES], out2[:N_NODES]], axis=-1)


# speedup vs baseline: 2.2286x; 2.2286x over previous
import jax
import jax.numpy as jnp
from jax import lax
from jax.experimental import pallas as pl
from jax.experimental.pallas import tpu as pltpu
from jax.experimental.pallas import tpu_sc as plsc

N_NODES = 10000
N_EDGES = 320000
D = 128
N_TILES = 32
NPT = 320
N_PAD = N_TILES * NPT
CHUNK = 3200
N_CHUNKS = N_EDGES // CHUNK
GB = 64
FBLK = 64


def _mm_body(d_ref, w_ref, b_ref, o_ref):
    o_ref[...] = jnp.maximum(
        jnp.dot(d_ref[...], w_ref[...], preferred_element_type=jnp.float32)
        + b_ref[...],
        0.0,
    )


def _encode(data_pad, W, b2):
    blk = 2048
    return pl.pallas_call(
        _mm_body,
        grid=(N_PAD // blk,),
        in_specs=[
            pl.BlockSpec((blk, D), lambda i: (i, 0)),
            pl.BlockSpec((D, D), lambda i: (0, 0)),
            pl.BlockSpec((1, D), lambda i: (0, 0)),
        ],
        out_specs=pl.BlockSpec((blk, D), lambda i: (i, 0)),
        out_shape=jax.ShapeDtypeStruct((N_PAD, D), jnp.float32),
    )(data_pad, W, b2)


def _edge_body(x_hbm, src_hbm, dst_hbm, out1_hbm, out2_hbm,
               acc, dstc, srcc, kdst, ksrc, rows, xblk, o1blk, o2blk,
               sem_d, sem_s, sem_g):
    c = lax.axis_index("c")
    s = lax.axis_index("s")
    t = s * 2 + c
    lo = t * NPT

    iota = lax.iota(jnp.int32, 16)
    neg = jnp.full((16,), -1.0, jnp.float32)
    npt_u = jnp.uint32(NPT)
    dnums = lax.GatherDimensionNumbers(
        offset_dims=(), collapsed_slice_dims=(0,), start_index_map=(0,))

    def init_row(r, _):
        for v in range(8):
            acc[r, pl.ds(v * 16, 16)] = neg
        return 0

    lax.fori_loop(0, NPT + 1, init_row, 0)

    def fire_edges(ci):
        base = ci * CHUNK
        buf = lax.rem(ci, 2)
        pltpu.async_copy(dst_hbm.at[pl.ds(base, CHUNK)], dstc.at[buf], sem_d)
        pltpu.async_copy(src_hbm.at[pl.ds(base, CHUNK)], srcc.at[buf], sem_s)

    fire_edges(0)

    def chunk_body(ci, _):
        buf = lax.rem(ci, 2)
        base = ci * CHUNK
        pltpu.make_async_copy(
            dst_hbm.at[pl.ds(base, CHUNK)], dstc.at[buf], sem_d).wait()
        pltpu.make_async_copy(
            src_hbm.at[pl.ds(base, CHUNK)], srcc.at[buf], sem_s).wait()

        @pl.when(ci + 1 < N_CHUNKS)
        def _():
            fire_edges(ci + 1)

        def filt(i, off):
            dvec = dstc[buf, pl.ds(i * 16, 16)]
            svec = srcc[buf, pl.ds(i * 16, 16)]
            dloc = dvec - lo
            m = plsc.bitcast(dloc, jnp.uint32) < npt_u
            mi = m.astype(jnp.int32)
            pos = off + plsc.cumsum(mi) - 1
            plsc.store_scatter(kdst, [pos], dloc, mask=m)
            plsc.store_scatter(ksrc, [pos], svec, mask=m)
            return off + plsc.all_reduce_population_count(m)

        off = lax.fori_loop(0, CHUNK // 16, filt, jnp.zeros((16,), jnp.int32))
        k = off[0]

        for j in range(GB // 16):
            kdst[pl.ds(k + j * 16, 16)] = jnp.full((16,), NPT, jnp.int32)
            ksrc[pl.ds(k + j * 16, 16)] = jnp.zeros((16,), jnp.int32)
        nb = (k + GB - 1) // GB

        def batch(b, _):
            pltpu.async_copy(
                x_hbm.at[ksrc.at[pl.ds(b * GB, GB)]], rows, sem_g).wait()

            def grp(g, _):
                e0 = b * GB + g * 16
                dvec = kdst[pl.ds(e0, 16)]
                for j in range(16):
                    dj = lax.gather(
                        dvec, jnp.full((16, 1), j, jnp.int32), dnums, (1,),
                        mode=lax.GatherScatterMode.PROMISE_IN_BOUNDS)
                    for v in range(8):
                        col = iota + v * 16
                        old = plsc.load_gather(acc, [dj, col])
                        new = jnp.maximum(old, rows[g * 16 + j,
                                                    pl.ds(v * 16, 16)])
                        plsc.store_scatter(acc, [dj, col], new)
                return 0

            lax.fori_loop(0, GB // 16, grp, 0)
            return 0

        lax.fori_loop(0, nb, batch, 0)
        return 0

    lax.fori_loop(0, N_CHUNKS, chunk_body, 0)

    def fin(bi, _):
        r0 = lo + bi * FBLK
        pltpu.sync_copy(x_hbm.at[pl.ds(r0, FBLK)], xblk)

        def frow(r, _):
            ar = bi * FBLK + r
            for v in range(8):
                sl = pl.ds(v * 16, 16)
                a = acc[ar, sl]
                xv = xblk[r, sl]
                valid = a >= 0.0
                o1blk[r, sl] = jnp.where(valid, xv, 0.0)
                o2blk[r, sl] = jnp.where(valid, a - xv, 0.0)
            return 0

        lax.fori_loop(0, FBLK, frow, 0)
        pltpu.sync_copy(o1blk, out1_hbm.at[pl.ds(r0, FBLK)])
        pltpu.sync_copy(o2blk, out2_hbm.at[pl.ds(r0, FBLK)])
        return 0

    lax.fori_loop(0, NPT // FBLK, fin, 0)


_edge_call = pl.kernel(
    _edge_body,
    out_type=[
        jax.ShapeDtypeStruct((N_PAD, D), jnp.float32),
        jax.ShapeDtypeStruct((N_PAD, D), jnp.float32),
    ],
    mesh=plsc.VectorSubcoreMesh(core_axis_name="c", subcore_axis_name="s"),
    compiler_params=pltpu.CompilerParams(needs_layout_passes=False),
    scratch_types=[
        pltpu.VMEM((NPT + 1, D), jnp.float32),
        pltpu.VMEM((2, CHUNK), jnp.int32),
        pltpu.VMEM((2, CHUNK), jnp.int32),
        pltpu.VMEM((CHUNK + GB,), jnp.int32),
        pltpu.VMEM((CHUNK + GB,), jnp.int32),
        pltpu.VMEM((GB, D), jnp.float32),
        pltpu.VMEM((FBLK, D), jnp.float32),
        pltpu.VMEM((FBLK, D), jnp.float32),
        pltpu.VMEM((FBLK, D), jnp.float32),
        pltpu.SemaphoreType.DMA,
        pltpu.SemaphoreType.DMA,
        pltpu.SemaphoreType.DMA,
    ],
)


def kernel(data, edge_index, W, b):
    data_pad = jnp.pad(data, ((0, N_PAD - N_NODES), (0, 0)))
    x = _encode(data_pad, W, b.reshape(1, D))
    src = edge_index[0]
    dst = edge_index[1]
    out1, out2 = _edge_call(x, src, dst)
    return jnp.concatenate([out1[:N_NODES], out2[:N_NODES]], axis=-1)


# speedup vs baseline: 4.2253x; 1.8959x over previous
import jax
import jax.numpy as jnp
from jax import lax
from jax.experimental import pallas as pl
from jax.experimental.pallas import tpu as pltpu
from jax.experimental.pallas import tpu_sc as plsc

N_NODES = 10000
N_EDGES = 320000
D = 128
DW = D // 2
N_TILES = 32
NPT = 320
N_PAD = N_TILES * NPT
CHUNK = 3200
N_CHUNKS = N_EDGES // CHUNK
GB = 64
FBLK = 64
NEG1_PAIR = -1082540160


def _mm_body(d_ref, w_ref, b_ref, o_ref, ob_ref):
    val = jnp.maximum(
        jnp.dot(d_ref[...], w_ref[...], preferred_element_type=jnp.float32)
        + b_ref[...],
        0.0,
    )
    o_ref[...] = val
    ob_ref[...] = val.astype(jnp.bfloat16)


def _encode(data_pad, W, b2):
    blk = 2048
    return pl.pallas_call(
        _mm_body,
        grid=(N_PAD // blk,),
        in_specs=[
            pl.BlockSpec((blk, D), lambda i: (i, 0)),
            pl.BlockSpec((D, D), lambda i: (0, 0)),
            pl.BlockSpec((1, D), lambda i: (0, 0)),
        ],
        out_specs=[
            pl.BlockSpec((blk, D), lambda i: (i, 0)),
            pl.BlockSpec((blk, D), lambda i: (i, 0)),
        ],
        out_shape=[
            jax.ShapeDtypeStruct((N_PAD, D), jnp.float32),
            jax.ShapeDtypeStruct((N_PAD, D), jnp.bfloat16),
        ],
    )(data_pad, W, b2)


def _edge_body(x_hbm, xb_hbm, src_hbm, dst_hbm, out1_hbm, out2_hbm,
               acc, dstc, srcc, kdst, ksrc, rows, xblk, o1blk, o2blk,
               sem_d, sem_s, sem_g):
    c = lax.axis_index("c")
    s = lax.axis_index("s")
    t = s * 2 + c
    lo = t * NPT

    iota = lax.iota(jnp.int32, 16)
    negp = jnp.full((16,), NEG1_PAIR, jnp.int32)
    npt_u = jnp.uint32(NPT)
    dnums = lax.GatherDimensionNumbers(
        offset_dims=(), collapsed_slice_dims=(0,), start_index_map=(0,))

    def init_row(r, _):
        for v in range(4):
            acc[r, pl.ds(v * 16, 16)] = negp
        return 0

    lax.fori_loop(0, NPT + 1, init_row, 0)

    def fire_edges(ci):
        base = ci * CHUNK
        buf = lax.rem(ci, 2)
        pltpu.async_copy(dst_hbm.at[pl.ds(base, CHUNK)], dstc.at[buf], sem_d)
        pltpu.async_copy(src_hbm.at[pl.ds(base, CHUNK)], srcc.at[buf], sem_s)

    fire_edges(0)

    def chunk_body(ci, _):
        buf = lax.rem(ci, 2)
        base = ci * CHUNK
        pltpu.make_async_copy(
            dst_hbm.at[pl.ds(base, CHUNK)], dstc.at[buf], sem_d).wait()
        pltpu.make_async_copy(
            src_hbm.at[pl.ds(base, CHUNK)], srcc.at[buf], sem_s).wait()

        @pl.when(ci + 1 < N_CHUNKS)
        def _():
            fire_edges(ci + 1)

        def filt(i, off):
            dvec = dstc[buf, pl.ds(i * 16, 16)]
            svec = srcc[buf, pl.ds(i * 16, 16)]
            dloc = dvec - lo
            m = plsc.bitcast(dloc, jnp.uint32) < npt_u
            mi = m.astype(jnp.int32)
            pos = off + plsc.cumsum(mi) - 1
            plsc.store_scatter(kdst, [pos], dloc, mask=m)
            plsc.store_scatter(ksrc, [pos], svec, mask=m)
            return off + plsc.all_reduce_population_count(m)

        off = lax.fori_loop(0, CHUNK // 16, filt, jnp.zeros((16,), jnp.int32))
        k = off[0]

        for j in range(GB // 16):
            kdst[pl.ds(k + j * 16, 16)] = jnp.full((16,), NPT, jnp.int32)
            ksrc[pl.ds(k + j * 16, 16)] = jnp.zeros((16,), jnp.int32)
        nb = (k + GB - 1) // GB

        def batch(b, _):
            pltpu.async_copy(
                xb_hbm.at[ksrc.at[pl.ds(b * GB, GB)]], rows, sem_g).wait()

            def grp(g, _):
                e0 = b * GB + g * 16
                dvec = kdst[pl.ds(e0, 16)]
                for j in range(16):
                    dj = lax.gather(
                        dvec, jnp.full((16, 1), j, jnp.int32), dnums, (1,),
                        mode=lax.GatherScatterMode.PROMISE_IN_BOUNDS)
                    for v in range(4):
                        col = iota + v * 16
                        old = plsc.bitcast(
                            plsc.load_gather(acc, [dj, col]), jnp.bfloat16)
                        rowv = plsc.bitcast(
                            rows[g * 16 + j, pl.ds(v * 16, 16)],
                            jnp.bfloat16)
                        new = jnp.maximum(old, rowv)
                        plsc.store_scatter(
                            acc, [dj, col], plsc.bitcast(new, jnp.int32))
                return 0

            lax.fori_loop(0, GB // 16, grp, 0)
            return 0

        lax.fori_loop(0, nb, batch, 0)
        return 0

    lax.fori_loop(0, N_CHUNKS, chunk_body, 0)

    def fin(bi, _):
        r0 = lo + bi * FBLK
        pltpu.sync_copy(x_hbm.at[pl.ds(r0, FBLK)], xblk)

        def frow(r, _):
            ar = bi * FBLK + r
            rsp = jnp.broadcast_to(r, (16,))
            for v in range(4):
                ab = plsc.bitcast(acc[ar, pl.ds(v * 16, 16)], jnp.bfloat16)
                a0, a1 = plsc.unpack(ab, format=plsc.PackFormat.INTERLEAVED)
                for h, ah in ((0, a0), (1, a1)):
                    cols = v * 32 + h + 2 * iota
                    xv = plsc.load_gather(xblk, [rsp, cols])
                    valid = ah >= 0.0
                    plsc.store_scatter(
                        o1blk, [rsp, cols], jnp.where(valid, xv, 0.0))
                    plsc.store_scatter(
                        o2blk, [rsp, cols], jnp.where(valid, ah - xv, 0.0))
            return 0

        lax.fori_loop(0, FBLK, frow, 0)
        pltpu.sync_copy(o1blk, out1_hbm.at[pl.ds(r0, FBLK)])
        pltpu.sync_copy(o2blk, out2_hbm.at[pl.ds(r0, FBLK)])
        return 0

    lax.fori_loop(0, NPT // FBLK, fin, 0)


_edge_call = pl.kernel(
    _edge_body,
    out_type=[
        jax.ShapeDtypeStruct((N_PAD, D), jnp.float32),
        jax.ShapeDtypeStruct((N_PAD, D), jnp.float32),
    ],
    mesh=plsc.VectorSubcoreMesh(core_axis_name="c", subcore_axis_name="s"),
    compiler_params=pltpu.CompilerParams(
        needs_layout_passes=False, use_tc_tiling_on_sc=False),
    scratch_types=[
        pltpu.VMEM((NPT + 1, DW), jnp.int32),
        pltpu.VMEM((2, CHUNK), jnp.int32),
        pltpu.VMEM((2, CHUNK), jnp.int32),
        pltpu.VMEM((CHUNK + GB,), jnp.int32),
        pltpu.VMEM((CHUNK + GB,), jnp.int32),
        pltpu.VMEM((GB, DW), jnp.int32),
        pltpu.VMEM((FBLK, D), jnp.float32),
        pltpu.VMEM((FBLK, D), jnp.float32),
        pltpu.VMEM((FBLK, D), jnp.float32),
        pltpu.SemaphoreType.DMA,
        pltpu.SemaphoreType.DMA,
        pltpu.SemaphoreType.DMA,
    ],
)


def kernel(data, edge_index, W, b):
    data_pad = jnp.pad(data, ((0, N_PAD - N_NODES), (0, 0)))
    x, xbf = _encode(data_pad, W, b.reshape(1, D))
    xb = jax.lax.bitcast_convert_type(
        xbf.reshape(N_PAD, DW, 2), jnp.int32)
    src = edge_index[0]
    dst = edge_index[1]
    out1, out2 = _edge_call(x, xb, src, dst)
    return jnp.concatenate([out1[:N_NODES], out2[:N_NODES]], axis=-1)
